# baseline (speedup 1.0000x reference)
import jax
import jax.numpy as jnp
from jax import lax
from jax.experimental import pallas as pl
from jax.experimental.pallas import tpu as pltpu
from jax.experimental.pallas import tpu_sc as plsc

_N = 50000
_E = 800000
_C = 121
_TEMP = 0.2

_NP = 50176
_EP = 819200
_EW = _EP // 32
_CH = 1600
_NCH = _EW // _CH
_NT = _NP // 16

_BLK = 2000

_f32 = jnp.float32
_i32 = jnp.int32

def _mk_mesh():
    return plsc.VectorSubcoreMesh(core_axis_name="c", subcore_axis_name="s",
                                  num_cores=2, num_subcores=16)


def _gate_body(src_h, trg_h, g0_h, g1_h, ub_h, vb_h, zeros_h,
               gate_h, degp_h,
               ub_v, vb_v, srcb, trgb, g0b, g1b, gateb, nb, deg_sh):
    c = lax.axis_index("c")
    s = lax.axis_index("s")
    w = c * 16 + s
    base = w * _EW
    pltpu.sync_copy(ub_h, ub_v)
    pltpu.sync_copy(vb_h, vb_v)
    pltpu.sync_copy(zeros_h, nb)
    pltpu.sync_copy(nb, deg_sh.at[pl.ds(s * _NT, _NT)])
    plsc.subcore_barrier()

    def chunk(ci, carry):
        off = pl.multiple_of(base + ci * _CH, _CH)
        pltpu.sync_copy(src_h.at[pl.ds(off, _CH)], srcb)
        pltpu.sync_copy(trg_h.at[pl.ds(off, _CH)], trgb)
        pltpu.sync_copy(g0_h.at[pl.ds(off, _CH)], g0b)
        pltpu.sync_copy(g1_h.at[pl.ds(off, _CH)], g1b)

        def grp(j, inner):
            jo = pl.multiple_of(j * 16, 16)
            sidx = srcb[pl.ds(jo, 16)]
            tidx = trgb[pl.ds(jo, 16)]
            a = plsc.load_gather(ub_v, [sidx])
            b = plsc.load_gather(vb_v, [tidx])
            e = (a + b) * 0.5
            p = 1.0 / (1.0 + jnp.exp(-e))
            z0 = (p + g0b[pl.ds(jo, 16)]) / _TEMP
            z1 = ((1.0 - p) + g1b[pl.ds(jo, 16)]) / _TEMP
            m = jnp.maximum(z0, z1)
            y0 = jnp.exp(z0 - m)
            y1 = jnp.exp(z1 - m)
            ss = y0 + y1
            g = jnp.where(y0 / ss >= y1 / ss, 1.0, 0.0).astype(_f32)
            gid = off + jo + lax.iota(_i32, 16)
            g = jnp.where(gid < _E, g, 0.0)
            gateb[pl.ds(jo, 16)] = g
            return inner

        lax.fori_loop(0, _CH // 16, grp, 0)
        pltpu.sync_copy(gateb, gate_h.at[pl.ds(off, _CH)])
        pltpu.sync_copy(gateb, deg_sh.at[srcb], add=True)
        return carry

    lax.fori_loop(0, _NCH, chunk, 0)
    plsc.subcore_barrier()
    pltpu.sync_copy(deg_sh.at[pl.ds(s * _NT, _NT)], nb)
    pltpu.sync_copy(nb, degp_h.at[pl.ds(c * _NP + s * _NT, _NT)])


def _sc_gate(src_p, trg_p, g0_p, g1_p, ub, vb, zeros1):
    return pl.kernel(
        _gate_body,
        out_type=(jax.ShapeDtypeStruct((_EP,), _f32),
                  jax.ShapeDtypeStruct((2 * _NP,), _f32)),
        mesh=_mk_mesh(),
        compiler_params=pltpu.CompilerParams(needs_layout_passes=False, use_tc_tiling_on_sc=False),
        scratch_types=[
            pltpu.VMEM((_NP,), _f32),
            pltpu.VMEM((_NP,), _f32),
            pltpu.VMEM((_CH,), _i32),
            pltpu.VMEM((_CH,), _i32),
            pltpu.VMEM((_CH,), _f32),
            pltpu.VMEM((_CH,), _f32),
            pltpu.VMEM((_CH,), _f32),
            pltpu.VMEM((_NT,), _f32),
            pltpu.VMEM_SHARED((_NP,), _f32),
        ],
    )(src_p, trg_p, g0_p, g1_p, ub, vb, zeros1)


_CH2 = 256
_ET = _EP // 16
_NC2 = _ET // _CH2
_PC = 8
_NTP = _NT // _PC

def _cheb_body(src_h, trg_h, gate_h, xt2_h, zeros_h,
               tx0_h, tx1_h,
               srcb, trgb, gateb, binb, gidxb, rv, nb, acc_sh, sem):
    c = lax.axis_index("c")
    s = lax.axis_index("s")

    pltpu.sync_copy(zeros_h, nb)
    for p in range(_PC):
        pltpu.sync_copy(nb, acc_sh.at[pl.ds(s * _NT + p * _NTP, _NTP), :])
    plsc.subcore_barrier()

    def chunk(ci, carry):
        off = pl.multiple_of(s * _ET + ci * _CH2, _CH2)
        pltpu.sync_copy(src_h.at[pl.ds(off, _CH2)], srcb)
        pltpu.sync_copy(trg_h.at[pl.ds(off, _CH2)], trgb)
        pltpu.sync_copy(gate_h.at[pl.ds(off, _CH2)], gateb)

        def grp(j, inner):
            jo = pl.multiple_of(j * 16, 16)
            sv = srcb[pl.ds(jo, 16)]
            tv = trgb[pl.ds(jo, 16)]
            gv = gateb[pl.ds(jo, 16)]
            binb[pl.ds(jo, 16)] = jnp.where(gv > 0.0, tv, _N)
            gidxb[pl.ds(jo, 16)] = sv + c * _NP
            return inner

        lax.fori_loop(0, _CH2 // 16, grp, 0)
        pltpu.async_copy(xt2_h.at[gidxb], rv, sem).wait()
        pltpu.sync_copy(rv, acc_sh.at[binb], add=True)
        return carry

    lax.fori_loop(0, _NC2, chunk, 0)
    plsc.subcore_barrier()
    for p in range(_PC):
        r0 = s * _NT + p * _NTP
        pltpu.sync_copy(acc_sh.at[pl.ds(r0, _NTP), :], nb)

        @pl.when(c == 0)
        def _():
            pltpu.sync_copy(nb, tx0_h.at[pl.ds(r0, _NTP), :])

        @pl.when(c == 1)
        def _():
            pltpu.sync_copy(nb, tx1_h.at[pl.ds(r0, _NTP), :])


def _sc_cheb(src_p, trg_p, gate_p, xt2, zeros2):
    return pl.kernel(
        _cheb_body,
        out_type=(jax.ShapeDtypeStruct((_NP, 32), _f32),
                  jax.ShapeDtypeStruct((_NP, 32), _f32)),
        mesh=_mk_mesh(),
        compiler_params=pltpu.CompilerParams(needs_layout_passes=False, use_tc_tiling_on_sc=False),
        scratch_types=[
            pltpu.VMEM((_CH2,), _i32),
            pltpu.VMEM((_CH2,), _i32),
            pltpu.VMEM((_CH2,), _f32),
            pltpu.VMEM((_CH2,), _i32),
            pltpu.VMEM((_CH2,), _i32),
            pltpu.VMEM((_CH2, 32), _f32),
            pltpu.VMEM((_NTP, 32), _f32),
            pltpu.VMEM_SHARED((_NP, 32), _f32),
            pltpu.SemaphoreType.DMA,
        ],
    )(src_p, trg_p, gate_p, xt2, zeros2)


_N2 = 2 * _NP
_AR = 102400
_ART = _AR // 16
_ARTP = _ART // _PC


def _att_body(src_h, trg_h, gate_h, es_h, et_h, zeros_h,
              daccp_h, bin4_h, gidx4_h,
              es_v, et_v, srcb, trgb, gateb, valb, binb, gidxb, nb, dacc_sh):
    c = lax.axis_index("c")
    s = lax.axis_index("s")
    w = c * 16 + s
    base = w * _EW
    pltpu.sync_copy(es_h, es_v)
    pltpu.sync_copy(et_h, et_v)
    pltpu.sync_copy(zeros_h, nb)
    pltpu.sync_copy(nb, dacc_sh.at[pl.ds(s * (_N2 // 16), _N2 // 16)])
    plsc.subcore_barrier()

    def chunk(ci, carry):
        off = pl.multiple_of(base + ci * _CH, _CH)
        pltpu.sync_copy(src_h.at[pl.ds(off, _CH)], srcb)
        pltpu.sync_copy(trg_h.at[pl.ds(off, _CH)], trgb)
        pltpu.sync_copy(gate_h.at[pl.ds(off, _CH)], gateb)

        def grp(j, inner):
            jo = pl.multiple_of(j * 16, 16)
            sv = srcb[pl.ds(jo, 16)]
            tv = trgb[pl.ds(jo, 16)]
            gv = gateb[pl.ds(jo, 16)]
            a = plsc.load_gather(es_v, [sv])
            b = plsc.load_gather(et_v, [tv])
            pos = (a + b) >= 0.0
            val = jnp.exp(jnp.where(pos, a, 0.2 * a))
            gid = off + jo + lax.iota(_i32, 16)
            val = jnp.where(gid < _E, val, 0.0)
            bofs = jnp.where(pos, 0, _NP)
            valb[pl.ds(jo, 16)] = val
            binb[pl.ds(jo, 16)] = tv + bofs
            gidxb[pl.ds(jo, 16)] = sv + bofs
            return inner

        lax.fori_loop(0, _CH // 16, grp, 0)
        pltpu.sync_copy(valb, dacc_sh.at[binb], add=True)
        pltpu.sync_copy(gidxb, gidx4_h.at[pl.ds(off, _CH)])

        def grp2(j, inner):
            jo = pl.multiple_of(j * 16, 16)
            gv = gateb[pl.ds(jo, 16)]
            bv = binb[pl.ds(jo, 16)]
            binb[pl.ds(jo, 16)] = jnp.where(gv > 0.0, bv, _N2)
            return inner

        lax.fori_loop(0, _CH // 16, grp2, 0)
        pltpu.sync_copy(binb, bin4_h.at[pl.ds(off, _CH)])
        return carry

    lax.fori_loop(0, _NCH, chunk, 0)
    plsc.subcore_barrier()
    pltpu.sync_copy(dacc_sh.at[pl.ds(s * (_N2 // 16), _N2 // 16)], nb)
    pltpu.sync_copy(nb, daccp_h.at[pl.ds(c * _N2 + s * (_N2 // 16), _N2 // 16)])


def _sc_att(src_p, trg_p, gate_p, es, et, zeros3):
    return pl.kernel(
        _att_body,
        out_type=(jax.ShapeDtypeStruct((2 * _N2,), _f32),
                  jax.ShapeDtypeStruct((_EP,), _i32),
                  jax.ShapeDtypeStruct((_EP,), _i32)),
        mesh=_mk_mesh(),
        compiler_params=pltpu.CompilerParams(needs_layout_passes=False, use_tc_tiling_on_sc=False),
        scratch_types=[
            pltpu.VMEM((_NP,), _f32),
            pltpu.VMEM((_NP,), _f32),
            pltpu.VMEM((_CH,), _i32),
            pltpu.VMEM((_CH,), _i32),
            pltpu.VMEM((_CH,), _f32),
            pltpu.VMEM((_CH,), _f32),
            pltpu.VMEM((_CH,), _i32),
            pltpu.VMEM((_CH,), _i32),
            pltpu.VMEM((_N2 // 16,), _f32),
            pltpu.VMEM_SHARED((_N2,), _f32),
        ],
    )(src_p, trg_p, gate_p, es, et, zeros3)


def _pred_body(gidx4_h, bin4_h, hpt_h, zeros_h,
               pacc_h,
               gidxb, binb, rv, nb, acc_sh, sem):
    c = lax.axis_index("c")
    s = lax.axis_index("s")

    for r in range(4):
        b = r * 2 + c
        pltpu.sync_copy(zeros_h, nb)
        for p in range(_PC):
            pltpu.sync_copy(
                nb, acc_sh.at[pl.ds(s * _ART + p * _ARTP, _ARTP), :])
        plsc.subcore_barrier()

        def chunk(ci, carry, b=b):
            off = pl.multiple_of(s * _ET + ci * _CH2, _CH2)
            pltpu.sync_copy(gidx4_h.at[pl.ds(off, _CH2)], gidxb)
            pltpu.sync_copy(bin4_h.at[pl.ds(off, _CH2)], binb)

            def grp(j, inner):
                jo = pl.multiple_of(j * 16, 16)
                gidxb[pl.ds(jo, 16)] = gidxb[pl.ds(jo, 16)] + b * _N2
                return inner

            lax.fori_loop(0, _CH2 // 16, grp, 0)
            pltpu.async_copy(hpt_h.at[gidxb], rv, sem).wait()
            pltpu.sync_copy(rv, acc_sh.at[binb], add=True)
            return carry

        lax.fori_loop(0, _NC2, chunk, 0)
        plsc.subcore_barrier()
        for p in range(_PC):
            r0 = s * _ART + p * _ARTP
            pltpu.sync_copy(acc_sh.at[pl.ds(r0, _ARTP), :], nb)
            pltpu.sync_copy(nb, pacc_h.at[pl.ds(b * _AR + r0, _ARTP), :])


def _sc_pred(gidx4_p, bin4_p, hpt, zeros4):
    return pl.kernel(
        _pred_body,
        out_type=jax.ShapeDtypeStruct((8 * _AR, 16), _f32),
        mesh=_mk_mesh(),
        compiler_params=pltpu.CompilerParams(needs_layout_passes=False, use_tc_tiling_on_sc=False),
        scratch_types=[
            pltpu.VMEM((_CH2,), _i32),
            pltpu.VMEM((_CH2,), _i32),
            pltpu.VMEM((_CH2, 16), _f32),
            pltpu.VMEM((_ARTP, 16), _f32),
            pltpu.VMEM_SHARED((_AR, 16), _f32),
            pltpu.SemaphoreType.DMA,
        ],
    )(gidx4_p, bin4_p, hpt, zeros4)


def _stage_a_body(x_ref, wp_ref, bp_ref, as_ref, at_ref, ub_ref, vb_ref):
    xb = x_ref[...]
    proj = jnp.dot(xb, wp_ref[...], preferred_element_type=_f32) + bp_ref[...]
    a0 = as_ref[0, :]
    a1 = as_ref[1, :]
    t0 = at_ref[0, :]
    t1 = at_ref[1, :]
    p0 = proj[:, :100]
    p1 = proj[:, 100:]
    ub_ref[...] = ((p0 * a0).sum(-1) + (p1 * a1).sum(-1))[:, None]
    vb_ref[...] = ((p0 * t0).sum(-1) + (p1 * t1).sum(-1))[:, None]


def _stage_a(x, W_proj_gate, b_proj_gate, a_src_gate, a_trg_gate):
    return pl.pallas_call(
        _stage_a_body,
        grid=(_N // _BLK,),
        in_specs=[
            pl.BlockSpec((_BLK, 64), lambda i: (i, 0)),
            pl.BlockSpec((64, 200), lambda i: (0, 0)),
            pl.BlockSpec((1, 200), lambda i: (0, 0)),
            pl.BlockSpec((2, 100), lambda i: (0, 0)),
            pl.BlockSpec((2, 100), lambda i: (0, 0)),
        ],
        out_specs=(pl.BlockSpec((_BLK, 1), lambda i: (i, 0)),
                   pl.BlockSpec((_BLK, 1), lambda i: (i, 0))),
        out_shape=(jax.ShapeDtypeStruct((_N, 1), _f32),
                   jax.ShapeDtypeStruct((_N, 1), _f32)),
    )(x, W_proj_gate, b_proj_gate.reshape(1, 200),
      a_src_gate.reshape(2, 100), a_trg_gate.reshape(2, 100))


def _stage_b_body(d0_ref, d1_ref, x_ref, dis_ref, xt_ref, gs_ref):
    i = pl.program_id(0)

    @pl.when(i == 0)
    def _():
        gs_ref[...] = jnp.zeros_like(gs_ref)

    deg = d0_ref[...] + d1_ref[...]
    dis = jnp.where(deg > 0,
                    1.0 / jnp.sqrt(jnp.maximum(deg, 1e-12)), 0.0)
    dis_ref[...] = dis
    xt_ref[...] = x_ref[...] * dis
    gs_ref[...] += deg.sum().reshape(1, 1)


def _stage_b(deg0, deg1, x):
    return pl.pallas_call(
        _stage_b_body,
        grid=(_N // _BLK,),
        in_specs=[
            pl.BlockSpec((_BLK, 1), lambda i: (i, 0)),
            pl.BlockSpec((_BLK, 1), lambda i: (i, 0)),
            pl.BlockSpec((_BLK, 64), lambda i: (i, 0)),
        ],
        out_specs=(pl.BlockSpec((_BLK, 1), lambda i: (i, 0)),
                   pl.BlockSpec((_BLK, 64), lambda i: (i, 0)),
                   pl.BlockSpec((1, 1), lambda i: (0, 0))),
        out_shape=(jax.ShapeDtypeStruct((_N, 1), _f32),
                   jax.ShapeDtypeStruct((_N, 64), _f32),
                   jax.ShapeDtypeStruct((1, 1), _f32)),
    )(deg0, deg1, x)


def _stage_c_body(x_ref, tx_ref, dis_ref, wsk_ref, bsk_ref, wc0_ref, wc1_ref,
                  bc_ref, wg_ref, asg_ref, atg_ref,
                  hpa_ref, hpc_ref, es_ref, et_ref, bb_ref, dd_ref):
    xb = x_ref[...]
    tx1 = (-dis_ref[...]) * tx_ref[...]
    xg = jax.nn.relu(
        jnp.dot(xb, wc0_ref[...], preferred_element_type=_f32)
        + jnp.dot(tx1, wc1_ref[...], preferred_element_type=_f32)
        + bc_ref[...])
    xs = jax.nn.relu(
        jnp.dot(xb, wsk_ref[...], preferred_element_type=_f32) + bsk_ref[...])
    h = xg + xs
    hp = jnp.dot(h, wg_ref[...], preferred_element_type=_f32)
    es = (hp * asg_ref[...]).sum(-1, keepdims=True)
    et = (hp * atg_ref[...]).sum(-1, keepdims=True)
    es_ref[...] = es
    et_ref[...] = et
    bb_ref[...] = jnp.exp(et)
    dd_ref[...] = jnp.exp(0.2 * et)
    hpad = jnp.concatenate([hp, jnp.zeros((hp.shape[0], 128 - _C), _f32)],
                           axis=1)
    hpa_ref[...] = hpad * jnp.exp(es)
    hpc_ref[...] = hpad * jnp.exp(0.2 * es)


def _stage_c(x, txcat, dis, W_skip, b_skip, W_cheb0, W_cheb1, b_cheb,
             W_gat, a_src_gat, a_trg_gat):
    return pl.pallas_call(
        _stage_c_body,
        grid=(_N // _BLK,),
        in_specs=[
            pl.BlockSpec((_BLK, 64), lambda i: (i, 0)),
            pl.BlockSpec((_BLK, 64), lambda i: (i, 0)),
            pl.BlockSpec((_BLK, 1), lambda i: (i, 0)),
            pl.BlockSpec((64, 300), lambda i: (0, 0)),
            pl.BlockSpec((1, 300), lambda i: (0, 0)),
            pl.BlockSpec((64, 300), lambda i: (0, 0)),
            pl.BlockSpec((64, 300), lambda i: (0, 0)),
            pl.BlockSpec((1, 300), lambda i: (0, 0)),
            pl.BlockSpec((300, _C), lambda i: (0, 0)),
            pl.BlockSpec((1, _C), lambda i: (0, 0)),
            pl.BlockSpec((1, _C), lambda i: (0, 0)),
        ],
        out_specs=(pl.BlockSpec((_BLK, 128), lambda i: (i, 0)),
                   pl.BlockSpec((_BLK, 128), lambda i: (i, 0)),
                   pl.BlockSpec((_BLK, 1), lambda i: (i, 0)),
                   pl.BlockSpec((_BLK, 1), lambda i: (i, 0)),
                   pl.BlockSpec((_BLK, 1), lambda i: (i, 0)),
                   pl.BlockSpec((_BLK, 1), lambda i: (i, 0))),
        out_shape=(jax.ShapeDtypeStruct((_N, 128), _f32),
                   jax.ShapeDtypeStruct((_N, 128), _f32),
                   jax.ShapeDtypeStruct((_N, 1), _f32),
                   jax.ShapeDtypeStruct((_N, 1), _f32),
                   jax.ShapeDtypeStruct((_N, 1), _f32),
                   jax.ShapeDtypeStruct((_N, 1), _f32)),
    )(x, txcat, dis, W_skip, b_skip.reshape(1, 300), W_cheb0, W_cheb1,
      b_cheb.reshape(1, 300), W_gat, a_src_gat.reshape(1, _C),
      a_trg_gat.reshape(1, _C))


def _stage_d_body(p_ref, q_ref, dp0_ref, dp1_ref, dn0_ref, dn1_ref,
                  bb_ref, dd_ref, bg_ref, ty_ref, mf_ref,
                  pred_ref, acc_ref):
    i = pl.program_id(0)

    @pl.when(i == 0)
    def _():
        acc_ref[...] = jnp.zeros_like(acc_ref)

    bb = bb_ref[...]
    dd = dd_ref[...]
    denom = bb * (dp0_ref[...] + dp1_ref[...]) \
        + dd * (dn0_ref[...] + dn1_ref[...])
    num = bb * p_ref[...] + dd * q_ref[...]
    pred = num[:, :_C] / (denom + 1e-16) + bg_ref[...]
    pred_ref[...] = pred
    ty = ty_ref[...]
    mf = mf_ref[...]
    bce = jnp.maximum(pred, 0.0) - pred * ty \
        + jnp.log1p(jnp.exp(-jnp.abs(pred)))
    acc_ref[...] += jnp.stack([(bce * mf).sum(), mf.sum()]).reshape(1, 2)


def _stage_d(P, Q, dp0, dp1, dn0, dn1, B, D, b_gat, true_y, maskf):
    return pl.pallas_call(
        _stage_d_body,
        grid=(_N // _BLK,),
        in_specs=[
            pl.BlockSpec((_BLK, 128), lambda i: (i, 0)),
            pl.BlockSpec((_BLK, 128), lambda i: (i, 0)),
            pl.BlockSpec((_BLK, 1), lambda i: (i, 0)),
            pl.BlockSpec((_BLK, 1), lambda i: (i, 0)),
            pl.BlockSpec((_BLK, 1), lambda i: (i, 0)),
            pl.BlockSpec((_BLK, 1), lambda i: (i, 0)),
            pl.BlockSpec((_BLK, 1), lambda i: (i, 0)),
            pl.BlockSpec((_BLK, 1), lambda i: (i, 0)),
            pl.BlockSpec((1, _C), lambda i: (0, 0)),
            pl.BlockSpec((_BLK, _C), lambda i: (i, 0)),
            pl.BlockSpec((_BLK, 1), lambda i: (i, 0)),
        ],
        out_specs=(pl.BlockSpec((_BLK, _C), lambda i: (i, 0)),
                   pl.BlockSpec((1, 2), lambda i: (0, 0))),
        out_shape=(jax.ShapeDtypeStruct((_N, _C), _f32),
                   jax.ShapeDtypeStruct((1, 2), _f32)),
    )(P, Q, dp0, dp1, dn0, dn1, B, D, b_gat.reshape(1, _C), true_y, maskf)


def _loss_body(pred_ref, ty_ref, mf_ref, acc_ref):
    i = pl.program_id(0)

    @pl.when(i == 0)
    def _():
        acc_ref[...] = jnp.zeros_like(acc_ref)

    z = pred_ref[...]
    ty = ty_ref[...]
    mf = mf_ref[...]
    bce = jnp.maximum(z, 0.0) - z * ty + jnp.log1p(jnp.exp(-jnp.abs(z)))
    acc_ref[...] += jnp.stack([(bce * mf).sum(), mf.sum()]).reshape(1, 2)


def _masked_bce_sums(pred, true_y, maskf):
    return pl.pallas_call(
        _loss_body,
        grid=(_N // _BLK,),
        in_specs=[
            pl.BlockSpec((_BLK, _C), lambda i: (i, 0)),
            pl.BlockSpec((_BLK, _C), lambda i: (i, 0)),
            pl.BlockSpec((_BLK, 1), lambda i: (i, 0)),
        ],
        out_specs=pl.BlockSpec((1, 2), lambda i: (0, 0)),
        out_shape=jax.ShapeDtypeStruct((1, 2), _f32),
    )(pred, true_y, maskf)


def kernel(x, edge_index, true_y, mask,
           W_proj_gate, b_proj_gate, a_src_gate, a_trg_gate,
           W_skip, b_skip, W_cheb0, W_cheb1, b_cheb,
           W_gat, a_src_gat, a_trg_gat, b_gat, gumbel_noise):
    src = edge_index[0]
    trg = edge_index[1]
    src_p = jnp.pad(src, (0, _EP - _E))
    trg_p = jnp.pad(trg, (0, _EP - _E))
    g0_p = jnp.pad(gumbel_noise[:, 0], (0, _EP - _E))
    g1_p = jnp.pad(gumbel_noise[:, 1], (0, _EP - _E))

    ub2, vb2 = _stage_a(x, W_proj_gate, b_proj_gate, a_src_gate, a_trg_gate)
    ub = jnp.pad(ub2[:, 0], (0, _NP - _N))
    vb = jnp.pad(vb2[:, 0], (0, _NP - _N))

    zeros1 = jnp.zeros((_NT,), _f32)
    gate_p, degp = _sc_gate(src_p, trg_p, g0_p, g1_p, ub, vb, zeros1)

    dis2, xt, gs = _stage_b(degp[:_N].reshape(-1, 1),
                            degp[_NP:_NP + _N].reshape(-1, 1), x)
    gate_sum = gs[0, 0]

    xt_pad = jnp.pad(xt, ((0, _NP - _N), (0, 0)))
    xt2 = jnp.concatenate([xt_pad[:, :32], xt_pad[:, 32:]], axis=0)
    zeros2 = jnp.zeros((_NTP, 32), _f32)
    tx0, tx1 = _sc_cheb(src_p, trg_p, gate_p, xt2, zeros2)
    txcat = jnp.concatenate([tx0[:_N], tx1[:_N]], axis=1)

    hpA, hpC, es2, et2, B2, D2 = _stage_c(
        x, txcat, dis2, W_skip, b_skip, W_cheb0, W_cheb1, b_cheb,
        W_gat, a_src_gat, a_trg_gat)

    es_pad = jnp.pad(es2[:, 0], (0, _NP - _N))
    et_pad = jnp.pad(et2[:, 0], (0, _NP - _N))
    zeros3 = jnp.zeros((_N2 // 16,), _f32)
    daccp, bin4_p, gidx4_p = _sc_att(src_p, trg_p, gate_p, es_pad, et_pad,
                                     zeros3)

    hpA_pad = jnp.pad(hpA, ((0, _NP - _N), (0, 0)))
    hpC_pad = jnp.pad(hpC, ((0, _NP - _N), (0, 0)))
    hpt = jnp.stack(
        [jnp.stack([hpA_pad[:, 16 * b:16 * b + 16],
                    hpC_pad[:, 16 * b:16 * b + 16]])
         for b in range(8)]).reshape(8 * _N2, 16)
    zeros4 = jnp.zeros((_ARTP, 16), _f32)
    pacc = _sc_pred(gidx4_p, bin4_p, hpt, zeros4).reshape(8, _AR, 16)

    P = jnp.concatenate([pacc[b, :_N, :] for b in range(8)], axis=1)
    Q = jnp.concatenate([pacc[b, _NP:_NP + _N, :] for b in range(8)], axis=1)
    pred, sums = _stage_d(
        P, Q, daccp[:_N].reshape(-1, 1),
        daccp[_N2:_N2 + _N].reshape(-1, 1),
        daccp[_NP:_NP + _N].reshape(-1, 1),
        daccp[_N2 + _NP:_N2 + _NP + _N].reshape(-1, 1),
        B2, D2, b_gat, true_y, mask.astype(_f32)[:, None])
    pred_loss = sums[0, 0] / (sums[0, 1] * _C) + 2.0 * gate_sum / _E
    return (pred_loss, pred)


# speedup vs baseline: 1.0348x; 1.0348x over previous
import jax
import jax.numpy as jnp
from jax import lax
from jax.experimental import pallas as pl
from jax.experimental.pallas import tpu as pltpu
from jax.experimental.pallas import tpu_sc as plsc

_N = 50000
_E = 800000
_C = 121
_TEMP = 0.2

_NP = 50176
_EP = 819200
_EW = _EP // 32
_CH = 1600
_NCH = _EW // _CH
_NT = _NP // 16

_BLK = 2000

_f32 = jnp.float32
_i32 = jnp.int32

def _mk_mesh():
    return plsc.VectorSubcoreMesh(core_axis_name="c", subcore_axis_name="s",
                                  num_cores=2, num_subcores=16)


def _gate_body(src_h, trg_h, g0_h, g1_h, ub_h, vb_h, zeros_h,
               gate_h, degp_h,
               ub_v, vb_v, srcb, trgb, g0b, g1b, gateb, nb, deg_sh):
    c = lax.axis_index("c")
    s = lax.axis_index("s")
    w = c * 16 + s
    base = w * _EW
    pltpu.sync_copy(ub_h, ub_v)
    pltpu.sync_copy(vb_h, vb_v)
    pltpu.sync_copy(zeros_h, nb)
    pltpu.sync_copy(nb, deg_sh.at[pl.ds(s * _NT, _NT)])
    plsc.subcore_barrier()

    def chunk(ci, carry):
        off = pl.multiple_of(base + ci * _CH, _CH)
        pltpu.sync_copy(src_h.at[pl.ds(off, _CH)], srcb)
        pltpu.sync_copy(trg_h.at[pl.ds(off, _CH)], trgb)
        pltpu.sync_copy(g0_h.at[pl.ds(off, _CH)], g0b)
        pltpu.sync_copy(g1_h.at[pl.ds(off, _CH)], g1b)

        def grp(j, inner):
            jo = pl.multiple_of(j * 16, 16)
            sidx = srcb[pl.ds(jo, 16)]
            tidx = trgb[pl.ds(jo, 16)]
            a = plsc.load_gather(ub_v, [sidx])
            b = plsc.load_gather(vb_v, [tidx])
            e = (a + b) * 0.5
            p = 1.0 / (1.0 + jnp.exp(-e))
            z0 = (p + g0b[pl.ds(jo, 16)]) / _TEMP
            z1 = ((1.0 - p) + g1b[pl.ds(jo, 16)]) / _TEMP
            m = jnp.maximum(z0, z1)
            y0 = jnp.exp(z0 - m)
            y1 = jnp.exp(z1 - m)
            ss = y0 + y1
            g = jnp.where(y0 / ss >= y1 / ss, 1.0, 0.0).astype(_f32)
            gid = off + jo + lax.iota(_i32, 16)
            g = jnp.where(gid < _E, g, 0.0)
            gateb[pl.ds(jo, 16)] = g
            return inner

        lax.fori_loop(0, _CH // 16, grp, 0)
        pltpu.sync_copy(gateb, gate_h.at[pl.ds(off, _CH)])
        pltpu.sync_copy(gateb, deg_sh.at[srcb], add=True)
        return carry

    lax.fori_loop(0, _NCH, chunk, 0)
    plsc.subcore_barrier()
    pltpu.sync_copy(deg_sh.at[pl.ds(s * _NT, _NT)], nb)
    pltpu.sync_copy(nb, degp_h.at[pl.ds(c * _NP + s * _NT, _NT)])


def _sc_gate(src_p, trg_p, g0_p, g1_p, ub, vb, zeros1):
    return pl.kernel(
        _gate_body,
        out_type=(jax.ShapeDtypeStruct((_EP,), _f32),
                  jax.ShapeDtypeStruct((2 * _NP,), _f32)),
        mesh=_mk_mesh(),
        compiler_params=pltpu.CompilerParams(needs_layout_passes=False, use_tc_tiling_on_sc=False),
        scratch_types=[
            pltpu.VMEM((_NP,), _f32),
            pltpu.VMEM((_NP,), _f32),
            pltpu.VMEM((_CH,), _i32),
            pltpu.VMEM((_CH,), _i32),
            pltpu.VMEM((_CH,), _f32),
            pltpu.VMEM((_CH,), _f32),
            pltpu.VMEM((_CH,), _f32),
            pltpu.VMEM((_NT,), _f32),
            pltpu.VMEM_SHARED((_NP,), _f32),
        ],
    )(src_p, trg_p, g0_p, g1_p, ub, vb, zeros1)


_CH2 = 256
_ET = _EP // 16
_NC2 = _ET // _CH2
_PC = 8
_NTP = _NT // _PC

def _cheb_body(src_h, trg_h, gate_h, xt2_h, zeros_h,
               tx0_h, tx1_h,
               srcb, trgb, gateb, binb, gidxb, rv, nb, acc_sh, sem):
    c = lax.axis_index("c")
    s = lax.axis_index("s")

    pltpu.sync_copy(zeros_h, nb)
    for p in range(_PC):
        pltpu.sync_copy(nb, acc_sh.at[pl.ds(s * _NT + p * _NTP, _NTP), :])
    plsc.subcore_barrier()

    def chunk(ci, carry):
        off = pl.multiple_of(s * _ET + ci * _CH2, _CH2)
        pltpu.sync_copy(src_h.at[pl.ds(off, _CH2)], srcb)
        pltpu.sync_copy(trg_h.at[pl.ds(off, _CH2)], trgb)
        pltpu.sync_copy(gate_h.at[pl.ds(off, _CH2)], gateb)

        def grp(j, inner):
            jo = pl.multiple_of(j * 16, 16)
            sv = srcb[pl.ds(jo, 16)]
            tv = trgb[pl.ds(jo, 16)]
            gv = gateb[pl.ds(jo, 16)]
            binb[pl.ds(jo, 16)] = jnp.where(gv > 0.0, tv, _N)
            gidxb[pl.ds(jo, 16)] = sv + c * _NP
            return inner

        lax.fori_loop(0, _CH2 // 16, grp, 0)
        pltpu.async_copy(xt2_h.at[gidxb], rv, sem).wait()
        pltpu.sync_copy(rv, acc_sh.at[binb], add=True)
        return carry

    lax.fori_loop(0, _NC2, chunk, 0)
    plsc.subcore_barrier()
    for p in range(_PC):
        r0 = s * _NT + p * _NTP
        pltpu.sync_copy(acc_sh.at[pl.ds(r0, _NTP), :], nb)

        @pl.when(c == 0)
        def _():
            pltpu.sync_copy(nb, tx0_h.at[pl.ds(r0, _NTP), :])

        @pl.when(c == 1)
        def _():
            pltpu.sync_copy(nb, tx1_h.at[pl.ds(r0, _NTP), :])


def _sc_cheb(src_p, trg_p, gate_p, xt2, zeros2):
    return pl.kernel(
        _cheb_body,
        out_type=(jax.ShapeDtypeStruct((_NP, 32), _f32),
                  jax.ShapeDtypeStruct((_NP, 32), _f32)),
        mesh=_mk_mesh(),
        compiler_params=pltpu.CompilerParams(needs_layout_passes=False, use_tc_tiling_on_sc=False),
        scratch_types=[
            pltpu.VMEM((_CH2,), _i32),
            pltpu.VMEM((_CH2,), _i32),
            pltpu.VMEM((_CH2,), _f32),
            pltpu.VMEM((_CH2,), _i32),
            pltpu.VMEM((_CH2,), _i32),
            pltpu.VMEM((_CH2, 32), _f32),
            pltpu.VMEM((_NTP, 32), _f32),
            pltpu.VMEM_SHARED((_NP, 32), _f32),
            pltpu.SemaphoreType.DMA,
        ],
    )(src_p, trg_p, gate_p, xt2, zeros2)


_N2 = 2 * _NP
_AR = 102400
_ART = _AR // 16
_ARTP = _ART // _PC


def _att_body(src_h, trg_h, gate_h, es_h, et_h, zeros_h,
              daccp_h, bin4_h, gidx4_h,
              es_v, et_v, srcb, trgb, gateb, valb, binb, gidxb, nb, dacc_sh):
    c = lax.axis_index("c")
    s = lax.axis_index("s")
    w = c * 16 + s
    base = w * _EW
    pltpu.sync_copy(es_h, es_v)
    pltpu.sync_copy(et_h, et_v)
    pltpu.sync_copy(zeros_h, nb)
    pltpu.sync_copy(nb, dacc_sh.at[pl.ds(s * (_N2 // 16), _N2 // 16)])
    plsc.subcore_barrier()

    def chunk(ci, carry):
        off = pl.multiple_of(base + ci * _CH, _CH)
        pltpu.sync_copy(src_h.at[pl.ds(off, _CH)], srcb)
        pltpu.sync_copy(trg_h.at[pl.ds(off, _CH)], trgb)
        pltpu.sync_copy(gate_h.at[pl.ds(off, _CH)], gateb)

        def grp(j, inner):
            jo = pl.multiple_of(j * 16, 16)
            sv = srcb[pl.ds(jo, 16)]
            tv = trgb[pl.ds(jo, 16)]
            gv = gateb[pl.ds(jo, 16)]
            a = plsc.load_gather(es_v, [sv])
            b = plsc.load_gather(et_v, [tv])
            pos = (a + b) >= 0.0
            val = jnp.exp(jnp.where(pos, a, 0.2 * a))
            gid = off + jo + lax.iota(_i32, 16)
            val = jnp.where(gid < _E, val, 0.0)
            bofs = jnp.where(pos, 0, _NP)
            valb[pl.ds(jo, 16)] = val
            binb[pl.ds(jo, 16)] = tv + bofs
            gidxb[pl.ds(jo, 16)] = sv + bofs
            return inner

        lax.fori_loop(0, _CH // 16, grp, 0)
        pltpu.sync_copy(valb, dacc_sh.at[binb], add=True)
        pltpu.sync_copy(gidxb, gidx4_h.at[pl.ds(off, _CH)])

        def grp2(j, inner):
            jo = pl.multiple_of(j * 16, 16)
            gv = gateb[pl.ds(jo, 16)]
            bv = binb[pl.ds(jo, 16)]
            binb[pl.ds(jo, 16)] = jnp.where(gv > 0.0, bv, _N2)
            return inner

        lax.fori_loop(0, _CH // 16, grp2, 0)
        pltpu.sync_copy(binb, bin4_h.at[pl.ds(off, _CH)])
        return carry

    lax.fori_loop(0, _NCH, chunk, 0)
    plsc.subcore_barrier()
    pltpu.sync_copy(dacc_sh.at[pl.ds(s * (_N2 // 16), _N2 // 16)], nb)
    pltpu.sync_copy(nb, daccp_h.at[pl.ds(c * _N2 + s * (_N2 // 16), _N2 // 16)])


def _sc_att(src_p, trg_p, gate_p, es, et, zeros3):
    return pl.kernel(
        _att_body,
        out_type=(jax.ShapeDtypeStruct((2 * _N2,), _f32),
                  jax.ShapeDtypeStruct((_EP,), _i32),
                  jax.ShapeDtypeStruct((_EP,), _i32)),
        mesh=_mk_mesh(),
        compiler_params=pltpu.CompilerParams(needs_layout_passes=False, use_tc_tiling_on_sc=False),
        scratch_types=[
            pltpu.VMEM((_NP,), _f32),
            pltpu.VMEM((_NP,), _f32),
            pltpu.VMEM((_CH,), _i32),
            pltpu.VMEM((_CH,), _i32),
            pltpu.VMEM((_CH,), _f32),
            pltpu.VMEM((_CH,), _f32),
            pltpu.VMEM((_CH,), _i32),
            pltpu.VMEM((_CH,), _i32),
            pltpu.VMEM((_N2 // 16,), _f32),
            pltpu.VMEM_SHARED((_N2,), _f32),
        ],
    )(src_p, trg_p, gate_p, es, et, zeros3)


def _pred_body(gidx4_h, bin4_h, hpt_h, zeros_h,
               pacc_h,
               gidx0, bin0, gidx1, bin1, rv0, rv1, nb, acc_sh, sem):
    c = lax.axis_index("c")
    s = lax.axis_index("s")

    def load_adj(ci, gidxb, binb, b):
        off = pl.multiple_of(s * _ET + ci * _CH2, _CH2)
        pltpu.sync_copy(gidx4_h.at[pl.ds(off, _CH2)], gidxb)
        pltpu.sync_copy(bin4_h.at[pl.ds(off, _CH2)], binb)

        def grp(j, inner):
            jo = pl.multiple_of(j * 16, 16)
            gidxb[pl.ds(jo, 16)] = gidxb[pl.ds(jo, 16)] + b * _N2
            return inner

        lax.fori_loop(0, _CH2 // 16, grp, 0)

    for r in range(4):
        b = r * 2 + c
        pltpu.sync_copy(zeros_h, nb)
        for p in range(_PC):
            pltpu.sync_copy(
                nb, acc_sh.at[pl.ds(s * _ART + p * _ARTP, _ARTP), :])
        plsc.subcore_barrier()

        load_adj(0, gidx0, bin0, b)
        pltpu.async_copy(hpt_h.at[gidx0], rv0, sem)

        def pair(k, carry, b=b):
            load_adj(2 * k + 1, gidx1, bin1, b)
            pltpu.async_copy(hpt_h.at[gidx1], rv1, sem)
            pltpu.make_async_copy(hpt_h.at[gidx0], rv0, sem).wait()
            pltpu.sync_copy(rv0, acc_sh.at[bin0], add=True)

            @pl.when(k < _NC2 // 2 - 1)
            def _():
                load_adj(2 * k + 2, gidx0, bin0, b)
                pltpu.async_copy(hpt_h.at[gidx0], rv0, sem)

            pltpu.make_async_copy(hpt_h.at[gidx1], rv1, sem).wait()
            pltpu.sync_copy(rv1, acc_sh.at[bin1], add=True)
            return carry

        lax.fori_loop(0, _NC2 // 2, pair, 0)
        plsc.subcore_barrier()
        for p in range(_PC):
            r0 = s * _ART + p * _ARTP
            pltpu.sync_copy(acc_sh.at[pl.ds(r0, _ARTP), :], nb)
            pltpu.sync_copy(nb, pacc_h.at[pl.ds(b * _AR + r0, _ARTP), :])


def _sc_pred(gidx4_p, bin4_p, hpt, zeros4):
    return pl.kernel(
        _pred_body,
        out_type=jax.ShapeDtypeStruct((8 * _AR, 16), _f32),
        mesh=_mk_mesh(),
        compiler_params=pltpu.CompilerParams(needs_layout_passes=False, use_tc_tiling_on_sc=False),
        scratch_types=[
            pltpu.VMEM((_CH2,), _i32),
            pltpu.VMEM((_CH2,), _i32),
            pltpu.VMEM((_CH2,), _i32),
            pltpu.VMEM((_CH2,), _i32),
            pltpu.VMEM((_CH2, 16), _f32),
            pltpu.VMEM((_CH2, 16), _f32),
            pltpu.VMEM((_ARTP, 16), _f32),
            pltpu.VMEM_SHARED((_AR, 16), _f32),
            pltpu.SemaphoreType.DMA,
        ],
    )(gidx4_p, bin4_p, hpt, zeros4)


def _stage_a_body(x_ref, wp_ref, bp_ref, as_ref, at_ref, ub_ref, vb_ref):
    xb = x_ref[...]
    proj = jnp.dot(xb, wp_ref[...], preferred_element_type=_f32) + bp_ref[...]
    a0 = as_ref[0, :]
    a1 = as_ref[1, :]
    t0 = at_ref[0, :]
    t1 = at_ref[1, :]
    p0 = proj[:, :100]
    p1 = proj[:, 100:]
    ub_ref[...] = ((p0 * a0).sum(-1) + (p1 * a1).sum(-1))[:, None]
    vb_ref[...] = ((p0 * t0).sum(-1) + (p1 * t1).sum(-1))[:, None]


def _stage_a(x, W_proj_gate, b_proj_gate, a_src_gate, a_trg_gate):
    return pl.pallas_call(
        _stage_a_body,
        grid=(_N // _BLK,),
        in_specs=[
            pl.BlockSpec((_BLK, 64), lambda i: (i, 0)),
            pl.BlockSpec((64, 200), lambda i: (0, 0)),
            pl.BlockSpec((1, 200), lambda i: (0, 0)),
            pl.BlockSpec((2, 100), lambda i: (0, 0)),
            pl.BlockSpec((2, 100), lambda i: (0, 0)),
        ],
        out_specs=(pl.BlockSpec((_BLK, 1), lambda i: (i, 0)),
                   pl.BlockSpec((_BLK, 1), lambda i: (i, 0))),
        out_shape=(jax.ShapeDtypeStruct((_N, 1), _f32),
                   jax.ShapeDtypeStruct((_N, 1), _f32)),
    )(x, W_proj_gate, b_proj_gate.reshape(1, 200),
      a_src_gate.reshape(2, 100), a_trg_gate.reshape(2, 100))


def _stage_b_body(d0_ref, d1_ref, x_ref, dis_ref, xt_ref, gs_ref):
    i = pl.program_id(0)

    @pl.when(i == 0)
    def _():
        gs_ref[...] = jnp.zeros_like(gs_ref)

    deg = d0_ref[...] + d1_ref[...]
    dis = jnp.where(deg > 0,
                    1.0 / jnp.sqrt(jnp.maximum(deg, 1e-12)), 0.0)
    dis_ref[...] = dis
    xt_ref[...] = x_ref[...] * dis
    gs_ref[...] += deg.sum().reshape(1, 1)


def _stage_b(deg0, deg1, x):
    return pl.pallas_call(
        _stage_b_body,
        grid=(_N // _BLK,),
        in_specs=[
            pl.BlockSpec((_BLK, 1), lambda i: (i, 0)),
            pl.BlockSpec((_BLK, 1), lambda i: (i, 0)),
            pl.BlockSpec((_BLK, 64), lambda i: (i, 0)),
        ],
        out_specs=(pl.BlockSpec((_BLK, 1), lambda i: (i, 0)),
                   pl.BlockSpec((_BLK, 64), lambda i: (i, 0)),
                   pl.BlockSpec((1, 1), lambda i: (0, 0))),
        out_shape=(jax.ShapeDtypeStruct((_N, 1), _f32),
                   jax.ShapeDtypeStruct((_N, 64), _f32),
                   jax.ShapeDtypeStruct((1, 1), _f32)),
    )(deg0, deg1, x)


def _stage_c_body(x_ref, tx_ref, dis_ref, wsk_ref, bsk_ref, wc0_ref, wc1_ref,
                  bc_ref, wg_ref, asg_ref, atg_ref,
                  hpa_ref, hpc_ref, es_ref, et_ref, bb_ref, dd_ref):
    xb = x_ref[...]
    tx1 = (-dis_ref[...]) * tx_ref[...]
    xg = jax.nn.relu(
        jnp.dot(xb, wc0_ref[...], preferred_element_type=_f32)
        + jnp.dot(tx1, wc1_ref[...], preferred_element_type=_f32)
        + bc_ref[...])
    xs = jax.nn.relu(
        jnp.dot(xb, wsk_ref[...], preferred_element_type=_f32) + bsk_ref[...])
    h = xg + xs
    hp = jnp.dot(h, wg_ref[...], preferred_element_type=_f32)
    es = (hp * asg_ref[...]).sum(-1, keepdims=True)
    et = (hp * atg_ref[...]).sum(-1, keepdims=True)
    es_ref[...] = es
    et_ref[...] = et
    bb_ref[...] = jnp.exp(et)
    dd_ref[...] = jnp.exp(0.2 * et)
    hpad = jnp.concatenate([hp, jnp.zeros((hp.shape[0], 128 - _C), _f32)],
                           axis=1)
    hpa_ref[...] = hpad * jnp.exp(es)
    hpc_ref[...] = hpad * jnp.exp(0.2 * es)


def _stage_c(x, txcat, dis, W_skip, b_skip, W_cheb0, W_cheb1, b_cheb,
             W_gat, a_src_gat, a_trg_gat):
    return pl.pallas_call(
        _stage_c_body,
        grid=(_N // _BLK,),
        in_specs=[
            pl.BlockSpec((_BLK, 64), lambda i: (i, 0)),
            pl.BlockSpec((_BLK, 64), lambda i: (i, 0)),
            pl.BlockSpec((_BLK, 1), lambda i: (i, 0)),
            pl.BlockSpec((64, 300), lambda i: (0, 0)),
            pl.BlockSpec((1, 300), lambda i: (0, 0)),
            pl.BlockSpec((64, 300), lambda i: (0, 0)),
            pl.BlockSpec((64, 300), lambda i: (0, 0)),
            pl.BlockSpec((1, 300), lambda i: (0, 0)),
            pl.BlockSpec((300, _C), lambda i: (0, 0)),
            pl.BlockSpec((1, _C), lambda i: (0, 0)),
            pl.BlockSpec((1, _C), lambda i: (0, 0)),
        ],
        out_specs=(pl.BlockSpec((_BLK, 128), lambda i: (i, 0)),
                   pl.BlockSpec((_BLK, 128), lambda i: (i, 0)),
                   pl.BlockSpec((_BLK, 1), lambda i: (i, 0)),
                   pl.BlockSpec((_BLK, 1), lambda i: (i, 0)),
                   pl.BlockSpec((_BLK, 1), lambda i: (i, 0)),
                   pl.BlockSpec((_BLK, 1), lambda i: (i, 0))),
        out_shape=(jax.ShapeDtypeStruct((_N, 128), _f32),
                   jax.ShapeDtypeStruct((_N, 128), _f32),
                   jax.ShapeDtypeStruct((_N, 1), _f32),
                   jax.ShapeDtypeStruct((_N, 1), _f32),
                   jax.ShapeDtypeStruct((_N, 1), _f32),
                   jax.ShapeDtypeStruct((_N, 1), _f32)),
    )(x, txcat, dis, W_skip, b_skip.reshape(1, 300), W_cheb0, W_cheb1,
      b_cheb.reshape(1, 300), W_gat, a_src_gat.reshape(1, _C),
      a_trg_gat.reshape(1, _C))


def _stage_d_body(p_ref, q_ref, dp0_ref, dp1_ref, dn0_ref, dn1_ref,
                  bb_ref, dd_ref, bg_ref, ty_ref, mf_ref,
                  pred_ref, acc_ref):
    i = pl.program_id(0)

    @pl.when(i == 0)
    def _():
        acc_ref[...] = jnp.zeros_like(acc_ref)

    bb = bb_ref[...]
    dd = dd_ref[...]
    denom = bb * (dp0_ref[...] + dp1_ref[...]) \
        + dd * (dn0_ref[...] + dn1_ref[...])
    num = bb * p_ref[...] + dd * q_ref[...]
    pred = num[:, :_C] / (denom + 1e-16) + bg_ref[...]
    pred_ref[...] = pred
    ty = ty_ref[...]
    mf = mf_ref[...]
    bce = jnp.maximum(pred, 0.0) - pred * ty \
        + jnp.log1p(jnp.exp(-jnp.abs(pred)))
    acc_ref[...] += jnp.stack([(bce * mf).sum(), mf.sum()]).reshape(1, 2)


def _stage_d(P, Q, dp0, dp1, dn0, dn1, B, D, b_gat, true_y, maskf):
    return pl.pallas_call(
        _stage_d_body,
        grid=(_N // _BLK,),
        in_specs=[
            pl.BlockSpec((_BLK, 128), lambda i: (i, 0)),
            pl.BlockSpec((_BLK, 128), lambda i: (i, 0)),
            pl.BlockSpec((_BLK, 1), lambda i: (i, 0)),
            pl.BlockSpec((_BLK, 1), lambda i: (i, 0)),
            pl.BlockSpec((_BLK, 1), lambda i: (i, 0)),
            pl.BlockSpec((_BLK, 1), lambda i: (i, 0)),
            pl.BlockSpec((_BLK, 1), lambda i: (i, 0)),
            pl.BlockSpec((_BLK, 1), lambda i: (i, 0)),
            pl.BlockSpec((1, _C), lambda i: (0, 0)),
            pl.BlockSpec((_BLK, _C), lambda i: (i, 0)),
            pl.BlockSpec((_BLK, 1), lambda i: (i, 0)),
        ],
        out_specs=(pl.BlockSpec((_BLK, _C), lambda i: (i, 0)),
                   pl.BlockSpec((1, 2), lambda i: (0, 0))),
        out_shape=(jax.ShapeDtypeStruct((_N, _C), _f32),
                   jax.ShapeDtypeStruct((1, 2), _f32)),
    )(P, Q, dp0, dp1, dn0, dn1, B, D, b_gat.reshape(1, _C), true_y, maskf)


def _loss_body(pred_ref, ty_ref, mf_ref, acc_ref):
    i = pl.program_id(0)

    @pl.when(i == 0)
    def _():
        acc_ref[...] = jnp.zeros_like(acc_ref)

    z = pred_ref[...]
    ty = ty_ref[...]
    mf = mf_ref[...]
    bce = jnp.maximum(z, 0.0) - z * ty + jnp.log1p(jnp.exp(-jnp.abs(z)))
    acc_ref[...] += jnp.stack([(bce * mf).sum(), mf.sum()]).reshape(1, 2)


def _masked_bce_sums(pred, true_y, maskf):
    return pl.pallas_call(
        _loss_body,
        grid=(_N // _BLK,),
        in_specs=[
            pl.BlockSpec((_BLK, _C), lambda i: (i, 0)),
            pl.BlockSpec((_BLK, _C), lambda i: (i, 0)),
            pl.BlockSpec((_BLK, 1), lambda i: (i, 0)),
        ],
        out_specs=pl.BlockSpec((1, 2), lambda i: (0, 0)),
        out_shape=jax.ShapeDtypeStruct((1, 2), _f32),
    )(pred, true_y, maskf)


def kernel(x, edge_index, true_y, mask,
           W_proj_gate, b_proj_gate, a_src_gate, a_trg_gate,
           W_skip, b_skip, W_cheb0, W_cheb1, b_cheb,
           W_gat, a_src_gat, a_trg_gat, b_gat, gumbel_noise):
    src = edge_index[0]
    trg = edge_index[1]
    src_p = jnp.pad(src, (0, _EP - _E))
    trg_p = jnp.pad(trg, (0, _EP - _E))
    g0_p = jnp.pad(gumbel_noise[:, 0], (0, _EP - _E))
    g1_p = jnp.pad(gumbel_noise[:, 1], (0, _EP - _E))

    ub2, vb2 = _stage_a(x, W_proj_gate, b_proj_gate, a_src_gate, a_trg_gate)
    ub = jnp.pad(ub2[:, 0], (0, _NP - _N))
    vb = jnp.pad(vb2[:, 0], (0, _NP - _N))

    zeros1 = jnp.zeros((_NT,), _f32)
    gate_p, degp = _sc_gate(src_p, trg_p, g0_p, g1_p, ub, vb, zeros1)

    dis2, xt, gs = _stage_b(degp[:_N].reshape(-1, 1),
                            degp[_NP:_NP + _N].reshape(-1, 1), x)
    gate_sum = gs[0, 0]

    xt_pad = jnp.pad(xt, ((0, _NP - _N), (0, 0)))
    xt2 = jnp.concatenate([xt_pad[:, :32], xt_pad[:, 32:]], axis=0)
    zeros2 = jnp.zeros((_NTP, 32), _f32)
    tx0, tx1 = _sc_cheb(src_p, trg_p, gate_p, xt2, zeros2)
    txcat = jnp.concatenate([tx0[:_N], tx1[:_N]], axis=1)

    hpA, hpC, es2, et2, B2, D2 = _stage_c(
        x, txcat, dis2, W_skip, b_skip, W_cheb0, W_cheb1, b_cheb,
        W_gat, a_src_gat, a_trg_gat)

    es_pad = jnp.pad(es2[:, 0], (0, _NP - _N))
    et_pad = jnp.pad(et2[:, 0], (0, _NP - _N))
    zeros3 = jnp.zeros((_N2 // 16,), _f32)
    daccp, bin4_p, gidx4_p = _sc_att(src_p, trg_p, gate_p, es_pad, et_pad,
                                     zeros3)

    hpA_pad = jnp.pad(hpA, ((0, _NP - _N), (0, 0)))
    hpC_pad = jnp.pad(hpC, ((0, _NP - _N), (0, 0)))
    hpt = jnp.stack(
        [jnp.stack([hpA_pad[:, 16 * b:16 * b + 16],
                    hpC_pad[:, 16 * b:16 * b + 16]])
         for b in range(8)]).reshape(8 * _N2, 16)
    zeros4 = jnp.zeros((_ARTP, 16), _f32)
    pacc = _sc_pred(gidx4_p, bin4_p, hpt, zeros4).reshape(8, _AR, 16)

    P = jnp.concatenate([pacc[b, :_N, :] for b in range(8)], axis=1)
    Q = jnp.concatenate([pacc[b, _NP:_NP + _N, :] for b in range(8)], axis=1)
    pred, sums = _stage_d(
        P, Q, daccp[:_N].reshape(-1, 1),
        daccp[_N2:_N2 + _N].reshape(-1, 1),
        daccp[_NP:_NP + _N].reshape(-1, 1),
        daccp[_N2 + _NP:_N2 + _NP + _N].reshape(-1, 1),
        B2, D2, b_gat, true_y, mask.astype(_f32)[:, None])
    pred_loss = sums[0, 0] / (sums[0, 1] * _C) + 2.0 * gate_sum / _E
    return (pred_loss, pred)
